# Initial kernel scaffold; baseline (speedup 1.0000x reference)
import jax
import jax.numpy as jnp
from jax.experimental import pallas as pl


def kernel(x):
    raise NotImplementedError("write your pallas kernel here")


# speedup vs baseline: 39.4906x; 39.4906x over previous
import functools
import math

import jax
import jax.numpy as jnp
from jax import lax
from jax.experimental import pallas as pl
from jax.experimental.pallas import tpu as pltpu

_K = 0.05
_K_PERCENT = 0.5


def _key_to_float(c):
    bits = jnp.where(c >= 0, c, c ^ jnp.int32(0x7FFFFFFF))
    return lax.bitcast_convert_type(bits, jnp.float32)


def _fused_kernel(x_ref, out_ref, *, k_row, k_glob):
    x = x_ref[...]
    b = x.shape[0]

    def count_ge_row(fc):
        return jnp.sum((x >= fc).astype(jnp.int32), axis=1, keepdims=True)

    c0 = count_ge_row(jnp.zeros((b, 1), jnp.float32))
    ans = jnp.where(c0 >= k_row, jnp.int32(0), jnp.int32(-(2**31)))

    def row_body(i, ans):
        bit = jnp.int32(2**30) >> i
        cand = ans | bit
        cnt = count_ge_row(_key_to_float(cand))
        return jnp.where(cnt >= k_row, cand, ans)

    ans = lax.fori_loop(0, 31, row_body, ans)
    thr = _key_to_float(ans)

    xs = jnp.where((x >= thr) & (x != 0.0), x, -jnp.inf)

    def count_ge_all(fc):
        return jnp.sum((xs >= fc).astype(jnp.int32))

    g0 = count_ge_all(jnp.float32(0.0))
    ans2 = jnp.where(g0 >= k_glob, jnp.int32(0), jnp.int32(-(2**31)))

    def glob_body(i, ans2):
        bit = jnp.int32(2**30) >> i
        cand = ans2 | bit
        cnt = count_ge_all(_key_to_float(cand))
        return jnp.where(cnt >= k_glob, cand, ans2)

    ans2 = lax.fori_loop(0, 31, glob_body, ans2)
    tg = _key_to_float(ans2)

    out_ref[...] = jnp.where(xs >= tg, xs, 0.0)


def kernel(x):
    b, n = x.shape
    k_row = math.ceil(_K * n)
    k_glob = math.ceil(_K_PERCENT * b * k_row)
    return pl.pallas_call(
        functools.partial(_fused_kernel, k_row=k_row, k_glob=k_glob),
        out_shape=jax.ShapeDtypeStruct((b, n), x.dtype),
        in_specs=[pl.BlockSpec(memory_space=pltpu.VMEM)],
        out_specs=pl.BlockSpec(memory_space=pltpu.VMEM),
    )(x)
